# baseline (speedup 1.0000x reference)
import functools

import jax
import jax.numpy as jnp
from jax.experimental import pallas as pl
from jax.experimental.pallas import tpu as pltpu

N_GROUPS = 8
BM = 4096


def _mm_kernel(x_ref, w_ref, o_ref):
    o_ref[...] = jnp.dot(
        x_ref[...], w_ref[0], preferred_element_type=jnp.float32
    )


def kernel(inputs, W):
    batch, size = inputs.shape
    n_disc, k, d = W.shape
    grid = (n_disc, batch // BM)

    return pl.pallas_call(
        _mm_kernel,
        grid=grid,
        in_specs=[
            pl.BlockSpec((BM, k), lambda g, i: (i, g)),
            pl.BlockSpec((1, k, d), lambda g, i: (g, 0, 0)),
        ],
        out_specs=pl.BlockSpec((BM, d), lambda g, i: (i, g)),
        out_shape=jax.ShapeDtypeStruct((batch, n_disc * d), jnp.float32),
        compiler_params=pltpu.CompilerParams(
            dimension_semantics=("arbitrary", "arbitrary"),
        ),
    )(inputs, W)


# speedup vs baseline: 1.0043x; 1.0043x over previous
import jax
import jax.numpy as jnp
from jax.experimental import pallas as pl
from jax.experimental.pallas import tpu as pltpu

N_GROUPS = 8
BM = 512


def _mm_kernel(x_ref, w_ref, o_ref):
    k = w_ref.shape[1]
    d = w_ref.shape[2]
    for g in range(N_GROUPS):
        o_ref[:, g * d:(g + 1) * d] = jnp.dot(
            x_ref[:, g * k:(g + 1) * k],
            w_ref[g],
            preferred_element_type=jnp.float32,
        )


def kernel(inputs, W):
    batch, size = inputs.shape
    n_disc, k, d = W.shape
    grid = (batch // BM,)

    return pl.pallas_call(
        _mm_kernel,
        grid=grid,
        in_specs=[
            pl.BlockSpec((BM, size), lambda i: (i, 0)),
            pl.BlockSpec((n_disc, k, d), lambda i: (0, 0, 0)),
        ],
        out_specs=pl.BlockSpec((BM, n_disc * d), lambda i: (i, 0)),
        out_shape=jax.ShapeDtypeStruct((batch, n_disc * d), jnp.float32),
        compiler_params=pltpu.CompilerParams(
            dimension_semantics=("arbitrary",),
        ),
    )(inputs, W)


# speedup vs baseline: 1.0195x; 1.0151x over previous
import jax
import jax.numpy as jnp
from jax.experimental import pallas as pl
from jax.experimental.pallas import tpu as pltpu

N_GROUPS = 8
BM = 256


def _mm_kernel(x_ref, w_ref, o_ref):
    k = w_ref.shape[1]
    d = w_ref.shape[2]
    for g in range(N_GROUPS):
        o_ref[:, g * d:(g + 1) * d] = jnp.dot(
            x_ref[:, g * k:(g + 1) * k],
            w_ref[g],
            preferred_element_type=jnp.float32,
        )


def kernel(inputs, W):
    batch, size = inputs.shape
    n_disc, k, d = W.shape
    grid = (batch // BM,)

    return pl.pallas_call(
        _mm_kernel,
        grid=grid,
        in_specs=[
            pl.BlockSpec((BM, size), lambda i: (i, 0)),
            pl.BlockSpec((n_disc, k, d), lambda i: (0, 0, 0)),
        ],
        out_specs=pl.BlockSpec((BM, n_disc * d), lambda i: (i, 0)),
        out_shape=jax.ShapeDtypeStruct((batch, n_disc * d), jnp.float32),
        compiler_params=pltpu.CompilerParams(
            dimension_semantics=("arbitrary",),
        ),
    )(inputs, W)
